# baseline (speedup 1.0000x reference)
import functools

import jax
import jax.numpy as jnp
from jax import lax
from jax.experimental import pallas as pl
from jax.experimental.pallas import tpu as pltpu
from jax.experimental.pallas import tpu_sc as plsc

SEQ = 200
BATCH = 1024
EMBED = 64
B = SEQ * BATCH

NC = 2
NS = 16
NW = NC * NS
BPW = B // NW
CHUNK = 128
NCHUNK = BPW // CHUNK

_mesh = plsc.VectorSubcoreMesh(core_axis_name="c", subcore_axis_name="s")


@functools.partial(
    pl.kernel,
    mesh=_mesh,
    out_type=jax.ShapeDtypeStruct((B, EMBED), jnp.float32),
    compiler_params=pltpu.CompilerParams(use_tc_tiling_on_sc=False),
    scratch_types=[
        pltpu.VMEM((NCHUNK, CHUNK), jnp.int32),
        pltpu.VMEM((CHUNK, EMBED), jnp.float32),
        pltpu.SemaphoreType.DMA,
    ],
)
def _gather_kernel(idx_hbm, table_hbm, out_hbm, idx_v, buf, gsem):
    wid = lax.axis_index("s") * NC + lax.axis_index("c")
    base = wid * BPW
    pltpu.sync_copy(idx_hbm.at[wid], idx_v)

    def body(j, carry):
        pltpu.async_copy(table_hbm.at[idx_v.at[j]], buf, gsem).wait()
        pltpu.sync_copy(buf, out_hbm.at[pl.ds(base + j * CHUNK, CHUNK)])
        return carry

    lax.fori_loop(0, NCHUNK, body, 0)


def kernel(inputs, inputs_len, table):
    del inputs_len
    idx = inputs.reshape(NW, NCHUNK, CHUNK)
    out = _gather_kernel(idx, table)
    return out.reshape(SEQ, BATCH, EMBED)


# speedup vs baseline: 1.0443x; 1.0443x over previous
import functools

import jax
import jax.numpy as jnp
from jax import lax
from jax.experimental import pallas as pl
from jax.experimental.pallas import tpu as pltpu
from jax.experimental.pallas import tpu_sc as plsc

SEQ = 200
BATCH = 1024
EMBED = 64
B = SEQ * BATCH

NC = 2
NS = 16
NW = NC * NS
BPW = B // NW
CHUNK = 128
NCHUNK = BPW // CHUNK

_mesh = plsc.VectorSubcoreMesh(core_axis_name="c", subcore_axis_name="s")


K = 5
NG = NCHUNK // K


@functools.partial(
    pl.kernel,
    mesh=_mesh,
    out_type=jax.ShapeDtypeStruct((B, EMBED), jnp.float32),
    compiler_params=pltpu.CompilerParams(use_tc_tiling_on_sc=False),
    scratch_types=[
        pltpu.VMEM((NCHUNK, CHUNK), jnp.int32),
        pltpu.VMEM((K, CHUNK, EMBED), jnp.float32),
        pltpu.VMEM((K, CHUNK, EMBED), jnp.float32),
        pltpu.SemaphoreType.DMA,
        pltpu.SemaphoreType.DMA,
        pltpu.SemaphoreType.DMA,
        pltpu.SemaphoreType.DMA,
    ],
)
def _gather_kernel(idx_hbm, table_hbm, out_hbm, idx_v, buf_a, buf_b,
                   gsem_a, gsem_b, wsem_a, wsem_b):
    wid = lax.axis_index("s") * NC + lax.axis_index("c")
    base = wid * BPW
    pltpu.sync_copy(idx_hbm.at[wid], idx_v)

    def fire_gathers(g, buf, gsem):
        for b in range(K):
            pltpu.async_copy(table_hbm.at[idx_v.at[g * K + b]], buf.at[b], gsem)

    def drain_then_write(g, buf, gsem, wsem):
        for b in range(K):
            pltpu.make_async_copy(table_hbm.at[idx_v.at[g * K + b]],
                                  buf.at[b], gsem).wait()
        for b in range(K):
            pltpu.async_copy(
                buf.at[b], out_hbm.at[pl.ds(base + (g * K + b) * CHUNK, CHUNK)],
                wsem)

    def drain_writes(g, buf, wsem):
        for b in range(K):
            pltpu.make_async_copy(
                buf.at[b], out_hbm.at[pl.ds(base + (g * K + b) * CHUNK, CHUNK)],
                wsem).wait()

    fire_gathers(0, buf_a, gsem_a)

    def body(i, carry):
        g0 = 2 * i
        fire_gathers(g0 + 1, buf_b, gsem_b)
        drain_then_write(g0, buf_a, gsem_a, wsem_a)
        drain_writes(g0, buf_a, wsem_a)

        @pl.when(g0 + 2 < NG)
        def _():
            fire_gathers(g0 + 2, buf_a, gsem_a)

        drain_then_write(g0 + 1, buf_b, gsem_b, wsem_b)
        drain_writes(g0 + 1, buf_b, wsem_b)
        return carry

    lax.fori_loop(0, NG // 2, body, 0)


def kernel(inputs, inputs_len, table):
    del inputs_len
    idx = inputs.reshape(NW, NCHUNK, CHUNK)
    out = _gather_kernel(idx, table)
    return out.reshape(SEQ, BATCH, EMBED)


# speedup vs baseline: 1.0468x; 1.0024x over previous
import functools

import jax
import jax.numpy as jnp
from jax import lax
from jax.experimental import pallas as pl
from jax.experimental.pallas import tpu as pltpu
from jax.experimental.pallas import tpu_sc as plsc

SEQ = 200
BATCH = 1024
EMBED = 64

NC = 2
NS = 16
NW = NC * NS
CHUNK = 128
CPR = BATCH // CHUNK
NCHUNK = SEQ * CPR // NW
ROWS_STAGED = NCHUNK // CPR + 1

K = 5
NG = NCHUNK // K

_mesh = plsc.VectorSubcoreMesh(core_axis_name="c", subcore_axis_name="s")


@functools.partial(
    pl.kernel,
    mesh=_mesh,
    out_type=jax.ShapeDtypeStruct((SEQ, BATCH, EMBED), jnp.float32),
    compiler_params=pltpu.CompilerParams(use_tc_tiling_on_sc=False),
    scratch_types=[
        pltpu.VMEM((ROWS_STAGED, BATCH), jnp.int32),
        pltpu.VMEM((K, CHUNK, EMBED), jnp.float32),
        pltpu.VMEM((K, CHUNK, EMBED), jnp.float32),
        pltpu.SemaphoreType.DMA,
        pltpu.SemaphoreType.DMA,
        pltpu.SemaphoreType.DMA,
        pltpu.SemaphoreType.DMA,
    ],
)
def _gather_kernel(idx_hbm, table_hbm, out_hbm, idx_v, buf_a, buf_b,
                   gsem_a, gsem_b, wsem_a, wsem_b):
    wid = lax.axis_index("s") * NC + lax.axis_index("c")
    chunk0 = wid * NCHUNK
    row0 = chunk0 // CPR
    pltpu.sync_copy(idx_hbm.at[pl.ds(row0, ROWS_STAGED)], idx_v)

    def chunk_coords(c):
        s = c // CPR
        off = (c % CPR) * CHUNK
        return s, off, s - row0

    def fire_gathers(g, buf, gsem):
        for b in range(K):
            c = chunk0 + g * K + b
            _, off, lr = chunk_coords(c)
            pltpu.async_copy(table_hbm.at[idx_v.at[lr, pl.ds(off, CHUNK)]],
                             buf.at[b], gsem)

    def drain_then_write(g, buf, gsem, wsem):
        for b in range(K):
            c = chunk0 + g * K + b
            _, off, lr = chunk_coords(c)
            pltpu.make_async_copy(table_hbm.at[idx_v.at[lr, pl.ds(off, CHUNK)]],
                                  buf.at[b], gsem).wait()
        for b in range(K):
            c = chunk0 + g * K + b
            s, off, _ = chunk_coords(c)
            pltpu.async_copy(buf.at[b], out_hbm.at[s, pl.ds(off, CHUNK)], wsem)

    def drain_writes(g, buf, wsem):
        for b in range(K):
            c = chunk0 + g * K + b
            s, off, _ = chunk_coords(c)
            pltpu.make_async_copy(buf.at[b], out_hbm.at[s, pl.ds(off, CHUNK)],
                                  wsem).wait()

    fire_gathers(0, buf_a, gsem_a)

    def body(i, carry):
        g0 = 2 * i
        fire_gathers(g0 + 1, buf_b, gsem_b)
        drain_then_write(g0, buf_a, gsem_a, wsem_a)
        drain_writes(g0, buf_a, wsem_a)

        @pl.when(g0 + 2 < NG)
        def _():
            fire_gathers(g0 + 2, buf_a, gsem_a)

        drain_then_write(g0 + 1, buf_b, gsem_b, wsem_b)
        drain_writes(g0 + 1, buf_b, wsem_b)
        return carry

    lax.fori_loop(0, NG // 2, body, 0)


def kernel(inputs, inputs_len, table):
    del inputs_len
    return _gather_kernel(inputs, table)
